# baseline (speedup 1.0000x reference)
import functools

import jax
import jax.numpy as jnp
from jax import lax
from jax.experimental import pallas as pl
from jax.experimental.pallas import tpu as pltpu
from jax.experimental.pallas import tpu_sc as plsc

VOCAB = 100000
N_EMBD = 768
BATCH = 4
SEQ_LEN = 2048

_LANES = 16
_NC = 2
_NS = 16
_NW = _NC * _NS

_S_PER_W = SEQ_LEN // _NW
_CHUNK = 32
_NH = _S_PER_W // _CHUNK
_NSLOT = 4
_UNROLL = 4
_ROW_SLICES = N_EMBD // _LANES

_CHUNKS = [(h, b) for h in range(_NH) for b in range(BATCH)]


def _emb_body(ids_hbm, table_hbm, pos_hbm, out_hbm,
              idx_v, tok_v, pos_v, gsem, osem, psem, isem):
    wid = lax.axis_index("s") * _NC + lax.axis_index("c")
    s_base = wid * _S_PER_W

    id_cps = []
    for b in range(BATCH):
        off = pl.multiple_of(b * SEQ_LEN + s_base, _S_PER_W)
        id_cps.append(pltpu.async_copy(ids_hbm.at[pl.ds(off, _S_PER_W)],
                                       idx_v.at[b], isem))
    pos_cp = [pltpu.async_copy(
        pos_hbm.at[pl.ds(pl.multiple_of(s_base, _S_PER_W), _CHUNK)],
        pos_v, psem)]
    for cp in id_cps:
        cp.wait()

    def chunk_off(c):
        h, b = _CHUNKS[c]
        return pl.multiple_of(b * SEQ_LEN + s_base + h * _CHUNK, _CHUNK)

    def issue_gather(c, slot):
        h, b = _CHUNKS[c]
        idx = idx_v.at[b, pl.ds(h * _CHUNK, _CHUNK)]
        return pltpu.async_copy(table_hbm.at[idx], tok_v.at[slot],
                                gsem.at[slot])

    n = len(_CHUNKS)
    gather_cp = [None] * _NSLOT
    out_cp = [None] * _NSLOT
    for c in range(min(2, n)):
        gather_cp[c] = issue_gather(c, c)
    for c in range(n):
        slot = c % _NSLOT
        h, b = _CHUNKS[c]
        if c + 2 < n:
            s2 = (c + 2) % _NSLOT
            if out_cp[s2] is not None:
                out_cp[s2].wait()
                out_cp[s2] = None
            gather_cp[s2] = issue_gather(c + 2, s2)
        if b == 0 and pos_cp:
            pos_cp.pop(0).wait()
        gather_cp[slot].wait()

        def add_row(r, _):
            for j in range(_ROW_SLICES):
                sl = pl.ds(j * _LANES, _LANES)
                tok_v[slot, r, sl] = tok_v[slot, r, sl] + pos_v[r, sl]
            return 0

        lax.fori_loop(0, _CHUNK, add_row, 0)
        if b == BATCH - 1 and h + 1 < _NH:
            s_off = pl.multiple_of(s_base + (h + 1) * _CHUNK, _CHUNK)
            pos_cp.append(pltpu.async_copy(pos_hbm.at[pl.ds(s_off, _CHUNK)],
                                           pos_v, psem))
        out_cp[slot] = pltpu.async_copy(tok_v.at[slot],
                                        out_hbm.at[pl.ds(chunk_off(c), _CHUNK)],
                                        osem.at[slot])
    for cp in out_cp:
        if cp is not None:
            cp.wait()


@jax.jit
def _emb_call(ids_flat, token_table, position_table):
    mesh = plsc.VectorSubcoreMesh(core_axis_name="c", subcore_axis_name="s")
    k = functools.partial(
        pl.kernel,
        out_type=jax.ShapeDtypeStruct((BATCH * SEQ_LEN, N_EMBD), jnp.float32),
        mesh=mesh,
        scratch_types=[
            pltpu.VMEM((BATCH, _S_PER_W), jnp.int32),
            pltpu.VMEM((_NSLOT, _CHUNK, N_EMBD), jnp.float32),
            pltpu.VMEM((_CHUNK, N_EMBD), jnp.float32),
            pltpu.SemaphoreType.DMA((_NSLOT,)),
            pltpu.SemaphoreType.DMA((_NSLOT,)),
            pltpu.SemaphoreType.DMA,
            pltpu.SemaphoreType.DMA,
        ],
    )(_emb_body)
    return k(ids_flat, token_table, position_table)


def kernel(input_ids, token_table, position_table):
    ids_flat = input_ids.reshape(-1).astype(jnp.int32)
    out = _emb_call(ids_flat, token_table, position_table)
    return out.reshape(BATCH, SEQ_LEN, N_EMBD)


# speedup vs baseline: 1.4935x; 1.4935x over previous
import functools

import jax
import jax.numpy as jnp
from jax import lax
from jax.experimental import pallas as pl
from jax.experimental.pallas import tpu as pltpu
from jax.experimental.pallas import tpu_sc as plsc

VOCAB = 100000
N_EMBD = 768
BATCH = 4
SEQ_LEN = 2048

_LANES = 16
_NC = 2
_NS = 16
_NW = _NC * _NS

_S_PER_W = SEQ_LEN // _NW
_CHUNK = 32
_NH = _S_PER_W // _CHUNK
_NSLOT = 4
_UNROLL = 4
_ROW_SLICES = N_EMBD // _LANES

_CHUNKS = [(h, b) for h in range(_NH) for b in range(BATCH)]


def _emb_body(ids_hbm, table_hbm, pos_hbm, out_hbm,
              idx_v, tok_v, pos_v, gsem, osem, psem, isem):
    wid = lax.axis_index("s") * _NC + lax.axis_index("c")
    s_base = wid * _S_PER_W

    id_cps = []
    for b in range(BATCH):
        off = pl.multiple_of(b * SEQ_LEN + s_base, _S_PER_W)
        id_cps.append(pltpu.async_copy(ids_hbm.at[pl.ds(off, _S_PER_W)],
                                       idx_v.at[b], isem))
    pos_cp = [pltpu.async_copy(
        pos_hbm.at[pl.ds(pl.multiple_of(s_base, _S_PER_W), _CHUNK)],
        pos_v, psem)]
    for cp in id_cps:
        cp.wait()

    def chunk_off(c):
        h, b = _CHUNKS[c]
        return pl.multiple_of(b * SEQ_LEN + s_base + h * _CHUNK, _CHUNK)

    def issue_gather(c, slot):
        h, b = _CHUNKS[c]
        idx = idx_v.at[b, pl.ds(h * _CHUNK, _CHUNK)]
        return pltpu.async_copy(table_hbm.at[idx], tok_v.at[slot],
                                gsem.at[slot])

    n = len(_CHUNKS)
    gather_cp = [None] * _NSLOT
    out_cp = [None] * _NSLOT
    for c in range(min(2, n)):
        gather_cp[c] = issue_gather(c, c)
    for c in range(n):
        slot = c % _NSLOT
        h, b = _CHUNKS[c]
        if c + 2 < n:
            s2 = (c + 2) % _NSLOT
            if out_cp[s2] is not None:
                out_cp[s2].wait()
                out_cp[s2] = None
            gather_cp[s2] = issue_gather(c + 2, s2)
        if b == 0 and pos_cp:
            pos_cp.pop(0).wait()
        gather_cp[slot].wait()

        @plsc.parallel_loop(0, _CHUNK)
        def add_row(r):
            for j in range(_ROW_SLICES):
                sl = pl.ds(j * _LANES, _LANES)
                plsc.addupdate(tok_v.at[slot, r, sl], pos_v[r, sl])
        if b == BATCH - 1 and h + 1 < _NH:
            s_off = pl.multiple_of(s_base + (h + 1) * _CHUNK, _CHUNK)
            pos_cp.append(pltpu.async_copy(pos_hbm.at[pl.ds(s_off, _CHUNK)],
                                           pos_v, psem))
        out_cp[slot] = pltpu.async_copy(tok_v.at[slot],
                                        out_hbm.at[pl.ds(chunk_off(c), _CHUNK)],
                                        osem.at[slot])
    for cp in out_cp:
        if cp is not None:
            cp.wait()


@jax.jit
def _emb_call(ids_flat, token_table, position_table):
    mesh = plsc.VectorSubcoreMesh(core_axis_name="c", subcore_axis_name="s")
    k = functools.partial(
        pl.kernel,
        out_type=jax.ShapeDtypeStruct((BATCH * SEQ_LEN, N_EMBD), jnp.float32),
        mesh=mesh,
        scratch_types=[
            pltpu.VMEM((BATCH, _S_PER_W), jnp.int32),
            pltpu.VMEM((_NSLOT, _CHUNK, N_EMBD), jnp.float32),
            pltpu.VMEM((_CHUNK, N_EMBD), jnp.float32),
            pltpu.SemaphoreType.DMA((_NSLOT,)),
            pltpu.SemaphoreType.DMA((_NSLOT,)),
            pltpu.SemaphoreType.DMA,
            pltpu.SemaphoreType.DMA,
        ],
    )(_emb_body)
    return k(ids_flat, token_table, position_table)


def kernel(input_ids, token_table, position_table):
    ids_flat = input_ids.reshape(-1).astype(jnp.int32)
    out = _emb_call(ids_flat, token_table, position_table)
    return out.reshape(BATCH, SEQ_LEN, N_EMBD)
